# baseline (speedup 1.0000x reference)
import functools

import jax
import jax.numpy as jnp
from jax.experimental import pallas as pl
from jax.experimental.pallas import tpu as pltpu

_L = 4
_KT = 2048


def _norms_body(cb_ref, cn_ref, *, k, d):
    kt = min(_KT, k)
    nkt = k // kt

    def norm_tile(kk, carry):
        cb = cb_ref[0, pl.ds(kk * kt, kt), :]
        cn_ref[0, 0, pl.ds(kk * kt, kt)] = jnp.sum(cb * cb, axis=1)
        return carry
    jax.lax.fori_loop(0, nkt, norm_tile, 0)


def _code_norms(codebooks):
    nl, k, d = codebooks.shape
    return pl.pallas_call(
        functools.partial(_norms_body, k=k, d=d),
        grid=(nl,),
        in_specs=[pl.BlockSpec((1, k, d), lambda l: (l, 0, 0))],
        out_specs=pl.BlockSpec((1, 1, k), lambda l: (l, 0, 0)),
        out_shape=jax.ShapeDtypeStruct((nl, 1, k), jnp.float32),
    )(codebooks)


def _rvq_body(x_ref, cb_ref, cn_ref, q_ref, idx_ref, *, m, k, d):
    kt = min(_KT, k)
    nkt = k // kt

    r = x_ref[...]
    q_acc = jnp.zeros_like(r)
    for l in range(_L):
        xx = jnp.sum(r * r, axis=1, keepdims=True)

        def score_tile(kk, carry):
            best_val, best_idx = carry
            cb = cb_ref[l, pl.ds(kk * kt, kt), :]
            scores = xx + cn_ref[l, 0, pl.ds(kk * kt, kt)][None, :] - 2.0 * (
                jax.lax.dot_general(
                    r.astype(jnp.bfloat16), cb.astype(jnp.bfloat16),
                    (((1,), (1,)), ((), ())),
                    preferred_element_type=jnp.float32))
            tval = jnp.min(scores, axis=1)
            targ = jnp.argmin(scores, axis=1).astype(jnp.int32) + kk * kt
            upd = tval < best_val
            return (jnp.where(upd, tval, best_val),
                    jnp.where(upd, targ, best_idx))

        init = (jnp.full((m,), jnp.inf, jnp.float32),
                jnp.zeros((m,), jnp.int32))
        _, idx = jax.lax.fori_loop(0, nkt, score_tile, init)
        if l == 0:
            idx_ref[0, 0, :] = idx

        def gather_tile(kk, q):
            cb = cb_ref[l, pl.ds(kk * kt, kt), :]
            oh = (jax.lax.broadcasted_iota(jnp.int32, (m, kt), 1)
                  == (idx[:, None] - kk * kt)).astype(jnp.float32)
            return q + jax.lax.dot_general(
                oh, cb, (((1,), (0,)), ((), ())),
                preferred_element_type=jnp.float32,
                precision=jax.lax.Precision.HIGHEST)

        q = jax.lax.fori_loop(0, nkt, gather_tile,
                              jnp.zeros((m, d), jnp.float32))
        r = r - q
        q_acc = q_acc + q
    q_ref[...] = q_acc


def _rvq(flat_x, codebooks, *, m):
    n, d = flat_x.shape
    _, k, _ = codebooks.shape
    cn = _code_norms(codebooks)
    grid = (n // m,)
    body = functools.partial(_rvq_body, m=m, k=k, d=d)
    q_flat, idx = pl.pallas_call(
        body,
        grid=grid,
        in_specs=[
            pl.BlockSpec((m, d), lambda i: (i, 0)),
            pl.BlockSpec((_L, k, d), lambda i: (0, 0, 0)),
            pl.BlockSpec((_L, 1, k), lambda i: (0, 0, 0)),
        ],
        out_specs=[
            pl.BlockSpec((m, d), lambda i: (i, 0)),
            pl.BlockSpec((1, 1, m), lambda i: (i, 0, 0)),
        ],
        out_shape=[
            jax.ShapeDtypeStruct((n, d), jnp.float32),
            jax.ShapeDtypeStruct((n // m, 1, m), jnp.int32),
        ],
    )(flat_x, codebooks, cn)
    return q_flat, idx


def kernel(x, codebooks):
    b, d, t = x.shape
    flat_x = jnp.transpose(x, (0, 2, 1)).reshape(b * t, d)
    q_flat, idx = _rvq(flat_x, codebooks, m=128)
    quantized = jnp.transpose(q_flat.reshape(b, t, d), (0, 2, 1))
    total_loss = jnp.asarray(0.0, dtype=jnp.float32)
    return quantized, idx.reshape(b, t), total_loss
